# baseline (speedup 1.0000x reference)
import jax
import jax.numpy as jnp
from jax import lax
from jax.experimental import pallas as pl
from jax.experimental.pallas import tpu as pltpu
from jax.experimental.pallas import tpu_sc as plsc

N_SPECIES = 4
ALPHA = 0.1
T = 512
NW = 32
GC = 128
EC = 128


def _celu(x):
    return jnp.where(x > 0, x, ALPHA * (jnp.exp(x / ALPHA) - 1.0))


def _dot(x, w):
    return jnp.dot(x.astype(jnp.bfloat16), w.astype(jnp.bfloat16),
                   preferred_element_type=jnp.float32)


def _mlp_body(ts_ref, x_ref, w1, b1, w2, b2, w3, b3, w4, b4, out_ref):
    x = x_ref[...]
    h = _celu(_dot(x, w1[0]) + b1[0, 0])
    h = _celu(_dot(h, w2[0]) + b2[0, 0])
    h = _celu(_dot(h, w3[0]) + b3[0, 0])
    out_ref[...] = _dot(h, w4[0]) + b4[0, 0]


def _sc_route_body(aev_hbm, pos_hbm, out_hbm,
                   idx0, idx1, rows0, rows1, sem0, sem1):
    wid = lax.axis_index("s") * 2 + lax.axis_index("c")
    rows_pw = aev_hbm.shape[0] // NW
    base = wid * rows_pw
    bufs = ((idx0, rows0, sem0), (idx1, rows1, sem1))
    nsteps = rows_pw // GC

    def load_and_scatter(j, iv, rv, sm):
        off = base + j * GC
        pltpu.sync_copy(pos_hbm.at[pl.ds(off, GC)], iv)
        pltpu.sync_copy(aev_hbm.at[pl.ds(off, GC)], rv)
        pltpu.async_copy(rv, out_hbm.at[iv], sm)

    for k, (iv, rv, sm) in enumerate(bufs):
        load_and_scatter(k, iv, rv, sm)

    def pair(jj, carry):
        for k, (iv, rv, sm) in enumerate(bufs):
            pltpu.make_async_copy(rv, out_hbm.at[iv], sm).wait()
            load_and_scatter(jj * 2 + k, iv, rv, sm)
        return carry

    lax.fori_loop(1, nsteps // 2, pair, 0)
    for iv, rv, sm in bufs:
        pltpu.make_async_copy(rv, out_hbm.at[iv], sm).wait()


def _sc_egather_body(e_hbm, pos_hbm, out_hbm, idx_v, e_v, sem):
    wid = lax.axis_index("s") * 2 + lax.axis_index("c")
    apw = out_hbm.shape[0] // NW
    base = wid * apw

    def step(j, carry):
        off = base + j * EC
        pltpu.sync_copy(pos_hbm.at[pl.ds(off, EC)], idx_v)
        pltpu.async_copy(e_hbm.at[idx_v], e_v, sem).wait()
        pltpu.sync_copy(e_v, out_hbm.at[pl.ds(off, EC)])
        return carry

    lax.fori_loop(0, apw // EC, step, 0)


def _rowsum_body(x_ref, out_ref):
    out_ref[...] = jnp.sum(x_ref[...], axis=1, keepdims=True)


def kernel(species, aev, params):
    b, a = species.shape
    aev_dim = aev.shape[-1]
    n = b * a
    nt = n // T + N_SPECIES
    big_l = nt * T

    w1 = jnp.stack([params[s][0][0] for s in range(N_SPECIES)])
    b1 = jnp.stack([params[s][0][1] for s in range(N_SPECIES)])[:, None, :]
    w2 = jnp.stack([params[s][1][0] for s in range(N_SPECIES)])
    b2 = jnp.stack([params[s][1][1] for s in range(N_SPECIES)])[:, None, :]
    w3 = jnp.stack([params[s][2][0] for s in range(N_SPECIES)])
    b3 = jnp.stack([params[s][2][1] for s in range(N_SPECIES)])[:, None, :]
    w4 = jnp.stack([params[s][3][0] for s in range(N_SPECIES)])
    b4 = jnp.stack([params[s][3][1] for s in range(N_SPECIES)])[:, :, None]

    sp_flat = species.reshape(-1).astype(jnp.int32)
    oh = (sp_flat[:, None] == jnp.arange(N_SPECIES, dtype=jnp.int32)[None, :])
    csum = jnp.cumsum(oh.astype(jnp.int32), axis=0)
    counts = csum[-1]
    segp = ((counts + T - 1) // T) * T
    p = jnp.concatenate([jnp.zeros((1,), jnp.int32), jnp.cumsum(segp)])
    pos = (jnp.sum(jnp.where(oh, p[None, :N_SPECIES], 0), axis=1)
           + jnp.sum(csum * oh, axis=1) - 1)
    tile_start = jnp.arange(nt, dtype=jnp.int32) * T
    tile_sp = jnp.minimum(
        jnp.sum((tile_start[:, None] >= p[None, 1:]).astype(jnp.int32), axis=1),
        N_SPECIES - 1)

    x2 = aev.reshape(n, aev_dim)

    x_sorted = pl.kernel(
        _sc_route_body,
        out_type=jax.ShapeDtypeStruct((big_l, aev_dim), jnp.float32),
        mesh=plsc.VectorSubcoreMesh(core_axis_name="c", subcore_axis_name="s", num_cores=2, num_subcores=16),
        scratch_types=[
            pltpu.VMEM((GC,), jnp.int32),
            pltpu.VMEM((GC,), jnp.int32),
            pltpu.VMEM((GC, aev_dim), jnp.float32),
            pltpu.VMEM((GC, aev_dim), jnp.float32),
            pltpu.SemaphoreType.DMA,
            pltpu.SemaphoreType.DMA,
        ],
    )(x2, pos)

    f1, f2, f3 = w1.shape[2], w2.shape[2], w3.shape[2]
    grid_spec = pltpu.PrefetchScalarGridSpec(
        num_scalar_prefetch=1,
        grid=(nt,),
        in_specs=[
            pl.BlockSpec((T, aev_dim), lambda i, ts: (i, 0)),
            pl.BlockSpec((1, aev_dim, f1), lambda i, ts: (ts[i], 0, 0)),
            pl.BlockSpec((1, 1, f1), lambda i, ts: (ts[i], 0, 0)),
            pl.BlockSpec((1, f1, f2), lambda i, ts: (ts[i], 0, 0)),
            pl.BlockSpec((1, 1, f2), lambda i, ts: (ts[i], 0, 0)),
            pl.BlockSpec((1, f2, f3), lambda i, ts: (ts[i], 0, 0)),
            pl.BlockSpec((1, 1, f3), lambda i, ts: (ts[i], 0, 0)),
            pl.BlockSpec((1, f3, 1), lambda i, ts: (ts[i], 0, 0)),
            pl.BlockSpec((1, 1, 1), lambda i, ts: (ts[i], 0, 0)),
        ],
        out_specs=pl.BlockSpec((T, 1), lambda i, ts: (i, 0)),
    )
    e_pad = pl.pallas_call(
        _mlp_body,
        grid_spec=grid_spec,
        out_shape=jax.ShapeDtypeStruct((big_l, 1), jnp.float32),
    )(tile_sp, x_sorted, w1, b1, w2, b2, w3, b3, w4, b4)

    e_nat = pl.kernel(
        _sc_egather_body,
        out_type=jax.ShapeDtypeStruct((n,), jnp.float32),
        mesh=plsc.VectorSubcoreMesh(core_axis_name="c", subcore_axis_name="s", num_cores=2, num_subcores=16),
        scratch_types=[
            pltpu.VMEM((EC,), jnp.int32),
            pltpu.VMEM((EC,), jnp.float32),
            pltpu.SemaphoreType.DMA,
        ],
    )(e_pad.reshape(big_l), pos)

    energies = pl.pallas_call(
        _rowsum_body,
        grid=(8,),
        in_specs=[pl.BlockSpec((b // 8, a), lambda i: (i, 0))],
        out_specs=pl.BlockSpec((b // 8, 1), lambda i: (i, 0)),
        out_shape=jax.ShapeDtypeStruct((b, 1), jnp.float32),
    )(e_nat.reshape(b, a))

    return (species, energies.reshape(b))


# speedup vs baseline: 1.1725x; 1.1725x over previous
import jax
import jax.numpy as jnp
from jax import lax
from jax.experimental import pallas as pl
from jax.experimental.pallas import tpu as pltpu
from jax.experimental.pallas import tpu_sc as plsc

N_SPECIES = 4
ALPHA = 0.1
T = 1024
NW = 32
GC = 128
EC = 128


def _celu(x):
    return jnp.where(x > 0, x, ALPHA * (jnp.exp(x / ALPHA) - 1.0))


def _dot(x, w):
    return jnp.dot(x.astype(jnp.bfloat16), w,
                   preferred_element_type=jnp.float32)


def _mlp_body(ts_ref, x_ref, w1, b1, w2, b2, w3, b3, w4, b4, out_ref):
    x = x_ref[...]
    h = _celu(_dot(x, w1[0]) + b1[0, 0])
    h = _celu(_dot(h, w2[0]) + b2[0, 0])
    h = _celu(_dot(h, w3[0]) + b3[0, 0])
    out_ref[...] = _dot(h, w4[0]) + b4[0, 0]


def _sc_route_body(aev_hbm, pos_hbm, out_hbm,
                   idx0, idx1, rows0, rows1, sem0, sem1):
    wid = lax.axis_index("s") * 2 + lax.axis_index("c")
    rows_pw = aev_hbm.shape[0] // NW
    base = wid * rows_pw
    bufs = ((idx0, rows0, sem0), (idx1, rows1, sem1))
    nsteps = rows_pw // GC

    def load_and_scatter(j, iv, rv, sm):
        off = base + j * GC
        pltpu.sync_copy(pos_hbm.at[pl.ds(off, GC)], iv)
        pltpu.sync_copy(aev_hbm.at[pl.ds(off, GC)], rv)
        pltpu.async_copy(rv, out_hbm.at[iv], sm)

    for k, (iv, rv, sm) in enumerate(bufs):
        load_and_scatter(k, iv, rv, sm)

    def pair(jj, carry):
        for k, (iv, rv, sm) in enumerate(bufs):
            pltpu.make_async_copy(rv, out_hbm.at[iv], sm).wait()
            load_and_scatter(jj * 2 + k, iv, rv, sm)
        return carry

    lax.fori_loop(1, nsteps // 2, pair, 0)
    for iv, rv, sm in bufs:
        pltpu.make_async_copy(rv, out_hbm.at[iv], sm).wait()


def _sc_egather_body(e_hbm, pos_hbm, out_hbm, idx_v, e_v, sem):
    wid = lax.axis_index("s") * 2 + lax.axis_index("c")
    apw = out_hbm.shape[0] // NW
    base = wid * apw

    def step(j, carry):
        off = base + j * EC
        pltpu.sync_copy(pos_hbm.at[pl.ds(off, EC)], idx_v)
        pltpu.async_copy(e_hbm.at[idx_v], e_v, sem).wait()
        pltpu.sync_copy(e_v, out_hbm.at[pl.ds(off, EC)])
        return carry

    lax.fori_loop(0, apw // EC, step, 0)


def _rowsum_body(x_ref, out_ref):
    out_ref[...] = jnp.sum(x_ref[...], axis=1, keepdims=True)


def kernel(species, aev, params):
    b, a = species.shape
    aev_dim = aev.shape[-1]
    n = b * a
    nt = n // T + N_SPECIES
    big_l = nt * T

    bf16 = jnp.bfloat16
    w1 = jnp.stack([params[s][0][0] for s in range(N_SPECIES)]).astype(bf16)
    b1 = jnp.stack([params[s][0][1] for s in range(N_SPECIES)])[:, None, :]
    w2 = jnp.stack([params[s][1][0] for s in range(N_SPECIES)]).astype(bf16)
    b2 = jnp.stack([params[s][1][1] for s in range(N_SPECIES)])[:, None, :]
    w3 = jnp.stack([params[s][2][0] for s in range(N_SPECIES)]).astype(bf16)
    b3 = jnp.stack([params[s][2][1] for s in range(N_SPECIES)])[:, None, :]
    w4 = jnp.stack([params[s][3][0] for s in range(N_SPECIES)]).astype(bf16)
    b4 = jnp.stack([params[s][3][1] for s in range(N_SPECIES)])[:, :, None]

    sp_flat = species.reshape(-1).astype(jnp.int32)
    oh = (sp_flat[:, None] == jnp.arange(N_SPECIES, dtype=jnp.int32)[None, :])
    csum = jnp.cumsum(oh.astype(jnp.int32), axis=0)
    counts = csum[-1]
    segp = ((counts + T - 1) // T) * T
    p = jnp.concatenate([jnp.zeros((1,), jnp.int32), jnp.cumsum(segp)])
    pos = (jnp.sum(jnp.where(oh, p[None, :N_SPECIES], 0), axis=1)
           + jnp.sum(csum * oh, axis=1) - 1)
    tile_start = jnp.arange(nt, dtype=jnp.int32) * T
    tile_sp = jnp.minimum(
        jnp.sum((tile_start[:, None] >= p[None, 1:]).astype(jnp.int32), axis=1),
        N_SPECIES - 1)

    x2 = aev.reshape(n, aev_dim)

    x_sorted = pl.kernel(
        _sc_route_body,
        out_type=jax.ShapeDtypeStruct((big_l, aev_dim), jnp.float32),
        mesh=plsc.VectorSubcoreMesh(core_axis_name="c", subcore_axis_name="s", num_cores=2, num_subcores=16),
        scratch_types=[
            pltpu.VMEM((GC,), jnp.int32),
            pltpu.VMEM((GC,), jnp.int32),
            pltpu.VMEM((GC, aev_dim), jnp.float32),
            pltpu.VMEM((GC, aev_dim), jnp.float32),
            pltpu.SemaphoreType.DMA,
            pltpu.SemaphoreType.DMA,
        ],
    )(x2, pos)

    f1, f2, f3 = w1.shape[2], w2.shape[2], w3.shape[2]
    grid_spec = pltpu.PrefetchScalarGridSpec(
        num_scalar_prefetch=1,
        grid=(nt,),
        in_specs=[
            pl.BlockSpec((T, aev_dim), lambda i, ts: (i, 0)),
            pl.BlockSpec((1, aev_dim, f1), lambda i, ts: (ts[i], 0, 0)),
            pl.BlockSpec((1, 1, f1), lambda i, ts: (ts[i], 0, 0)),
            pl.BlockSpec((1, f1, f2), lambda i, ts: (ts[i], 0, 0)),
            pl.BlockSpec((1, 1, f2), lambda i, ts: (ts[i], 0, 0)),
            pl.BlockSpec((1, f2, f3), lambda i, ts: (ts[i], 0, 0)),
            pl.BlockSpec((1, 1, f3), lambda i, ts: (ts[i], 0, 0)),
            pl.BlockSpec((1, f3, 1), lambda i, ts: (ts[i], 0, 0)),
            pl.BlockSpec((1, 1, 1), lambda i, ts: (ts[i], 0, 0)),
        ],
        out_specs=pl.BlockSpec((T, 1), lambda i, ts: (i, 0)),
    )
    e_pad = pl.pallas_call(
        _mlp_body,
        grid_spec=grid_spec,
        out_shape=jax.ShapeDtypeStruct((big_l, 1), jnp.float32),
    )(tile_sp, x_sorted, w1, b1, w2, b2, w3, b3, w4, b4)

    e_nat = pl.kernel(
        _sc_egather_body,
        out_type=jax.ShapeDtypeStruct((n,), jnp.float32),
        mesh=plsc.VectorSubcoreMesh(core_axis_name="c", subcore_axis_name="s", num_cores=2, num_subcores=16),
        scratch_types=[
            pltpu.VMEM((EC,), jnp.int32),
            pltpu.VMEM((EC,), jnp.float32),
            pltpu.SemaphoreType.DMA,
        ],
    )(e_pad.reshape(big_l), pos)

    energies = pl.pallas_call(
        _rowsum_body,
        grid=(8,),
        in_specs=[pl.BlockSpec((b // 8, a), lambda i: (i, 0))],
        out_specs=pl.BlockSpec((b // 8, 1), lambda i: (i, 0)),
        out_shape=jax.ShapeDtypeStruct((b, 1), jnp.float32),
    )(e_nat.reshape(b, a))

    return (species, energies.reshape(b))
